# Initial kernel scaffold; baseline (speedup 1.0000x reference)
import functools

import jax
import jax.numpy as jnp
from jax.experimental import pallas as pl
from jax.experimental.pallas import tpu as pltpu


def _cheb_fused_body(g_ref, x_ref, w_ref, b_ref, out_ref, *, n, n_orders):
    f32 = jnp.float32
    bf16 = jnp.bfloat16

    g = g_ref[...]
    ones = jnp.ones((n, 128), dtype=bf16)
    deg = jnp.dot(g, ones, preferred_element_type=f32)[:, :1]
    dsc = jax.lax.rsqrt(deg)

    def s_matvec(xv):
        xs = (dsc * xv).astype(bf16)
        return dsc * jnp.dot(g, xs, preferred_element_type=f32)

    x0 = x_ref[...]
    out = jnp.dot(x0.astype(bf16), w_ref[0],
                  preferred_element_type=f32) + b_ref[...]

    x1 = x0 - s_matvec(x0)
    out += jnp.dot(x1.astype(bf16), w_ref[1], preferred_element_type=f32)

    xm2, xm1 = x0, x1
    for k in range(2, n_orders):
        xk = 2.0 * (xm1 - s_matvec(xm1)) - xm2
        out += jnp.dot(xk.astype(bf16), w_ref[k], preferred_element_type=f32)
        xm2, xm1 = xm1, xk

    out_ref[...] = out


def kernel(inputs, graph, weight, bias):
    f32 = jnp.float32
    bf16 = jnp.bfloat16

    x = jnp.asarray(inputs, f32)
    batch, n, c_in = x.shape
    w = jnp.asarray(weight, f32)[:, 0]
    n_orders, _, c_out = w.shape
    b_vec = jnp.asarray(bias, f32).reshape(1, c_out)

    g_bf = jnp.asarray(graph, f32).astype(bf16)

    half = batch // 2
    w_bd = jnp.einsum('be,kcd->kbced', jnp.eye(half, dtype=f32), w)
    w_bd = w_bd.reshape(n_orders, half * c_in, half * c_out).astype(bf16)
    b_bd = jnp.tile(b_vec, (1, half))

    x2d = x.transpose(1, 0, 2).reshape(n, batch * c_in)

    hc_in = half * c_in
    hc_out = half * c_out

    out2d = pl.pallas_call(
        functools.partial(_cheb_fused_body, n=n, n_orders=n_orders),
        out_shape=jax.ShapeDtypeStruct((n, batch * c_out), f32),
        grid=(2,),
        in_specs=[
            pl.BlockSpec((n, n), lambda c: (0, 0)),
            pl.BlockSpec((n, hc_in), lambda c: (0, c)),
            pl.BlockSpec((n_orders, hc_in, hc_out), lambda c: (0, 0, 0)),
            pl.BlockSpec((1, hc_out), lambda c: (0, 0)),
        ],
        out_specs=pl.BlockSpec((n, hc_out), lambda c: (0, c)),
        compiler_params=pltpu.CompilerParams(
            dimension_semantics=("core_parallel",),
            vmem_limit_bytes=64 * 1024 * 1024,
        ),
    )(g_bf, x2d, w_bd, b_bd)

    return out2d.reshape(n, batch, c_out).transpose(1, 0, 2)


# speedup vs baseline: 1.9280x; 1.9280x over previous
import functools

import jax
import jax.numpy as jnp
from jax.experimental import pallas as pl
from jax.experimental.pallas import tpu as pltpu


def _cheb_fused_body(g_ref, x_ref, w_ref, b_ref, out_ref, dsc_ref, *,
                     n, n_orders):
    f32 = jnp.float32
    bf16 = jnp.bfloat16

    g = g_ref[...]

    @pl.when(pl.program_id(0) == 0)
    def _():
        ones = jnp.ones((n, 128), dtype=bf16)
        deg = jnp.dot(g, ones, preferred_element_type=f32)[:, :1]
        dsc_ref[...] = jax.lax.rsqrt(deg)

    dsc = dsc_ref[...]

    def s_matvec(xv):
        xs = (dsc * xv).astype(bf16)
        return dsc * jnp.dot(g, xs, preferred_element_type=f32)

    x0 = x_ref[...]
    out = jnp.dot(x0.astype(bf16), w_ref[0],
                  preferred_element_type=f32) + b_ref[...]

    x1 = x0 - s_matvec(x0)
    out += jnp.dot(x1.astype(bf16), w_ref[1], preferred_element_type=f32)

    xm2, xm1 = x0, x1
    for k in range(2, n_orders):
        xk = 2.0 * (xm1 - s_matvec(xm1)) - xm2
        out += jnp.dot(xk.astype(bf16), w_ref[k], preferred_element_type=f32)
        xm2, xm1 = xm1, xk

    out_ref[...] = out


def kernel(inputs, graph, weight, bias):
    f32 = jnp.float32
    bf16 = jnp.bfloat16

    x = jnp.asarray(inputs, f32)
    batch, n, c_in = x.shape
    w = jnp.asarray(weight, f32)[:, 0]
    n_orders, _, c_out = w.shape
    b_vec = jnp.asarray(bias, f32).reshape(1, c_out)

    g_bf = jnp.asarray(graph, f32).astype(bf16)

    half = batch // 2
    w_bd = jnp.einsum('be,kcd->kbced', jnp.eye(half, dtype=f32), w)
    w_bd = w_bd.reshape(n_orders, half * c_in, half * c_out).astype(bf16)
    b_bd = jnp.tile(b_vec, (1, half))

    x2d = x.transpose(1, 0, 2).reshape(n, batch * c_in)

    hc_in = half * c_in
    hc_out = half * c_out

    out2d = pl.pallas_call(
        functools.partial(_cheb_fused_body, n=n, n_orders=n_orders),
        out_shape=jax.ShapeDtypeStruct((n, batch * c_out), f32),
        grid=(2,),
        in_specs=[
            pl.BlockSpec((n, n), lambda c: (0, 0)),
            pl.BlockSpec((n, hc_in), lambda c: (0, c)),
            pl.BlockSpec((n_orders, hc_in, hc_out), lambda c: (0, 0, 0)),
            pl.BlockSpec((1, hc_out), lambda c: (0, 0)),
        ],
        out_specs=pl.BlockSpec((n, hc_out), lambda c: (0, c)),
        scratch_shapes=[pltpu.VMEM((n, 1), f32)],
        compiler_params=pltpu.CompilerParams(
            dimension_semantics=("arbitrary",),
            vmem_limit_bytes=56 * 1024 * 1024,
        ),
    )(g_bf, x2d, w_bd, b_bd)

    return out2d.reshape(n, batch, c_out).transpose(1, 0, 2)


# speedup vs baseline: 2.2311x; 1.1572x over previous
import functools
import math

import jax
import jax.numpy as jnp
from jax.experimental import pallas as pl
from jax.experimental.pallas import tpu as pltpu


def _prep_body(g_ref, gbf_ref, dsc_ref):
    gf = g_ref[...]
    gbf_ref[...] = gf.astype(jnp.bfloat16)
    dsc_ref[...] = jax.lax.rsqrt(jnp.sum(gf, axis=1, keepdims=True))


def _cheb_body(g_ref, dsc_ref, x_ref, w_ref, b_ref, out_ref, *,
               n_orders, nb, c_out):
    f32 = jnp.float32
    bf16 = jnp.bfloat16

    g = g_ref[...]
    dsc = dsc_ref[...]

    def s_matvec(xv):
        xs = (dsc * xv).astype(bf16)
        return dsc * jnp.dot(g, xs, preferred_element_type=f32)

    x0 = x_ref[...]
    out = jnp.dot(x0.astype(bf16), w_ref[0],
                  preferred_element_type=f32) + b_ref[...]

    x1 = x0 - s_matvec(x0)
    out += jnp.dot(x1.astype(bf16), w_ref[1], preferred_element_type=f32)

    xm2, xm1 = x0, x1
    for k in range(2, n_orders):
        xk = 2.0 * (xm1 - s_matvec(xm1)) - xm2
        out += jnp.dot(xk.astype(bf16), w_ref[k], preferred_element_type=f32)
        xm2, xm1 = xm1, xk

    for b in range(nb):
        out_ref[b] = out[:, b * c_out:(b + 1) * c_out]


def kernel(inputs, graph, weight, bias):
    f32 = jnp.float32
    bf16 = jnp.bfloat16

    x = jnp.asarray(inputs, f32)
    batch, n, c_in = x.shape
    w = jnp.asarray(weight, f32)[:, 0]
    n_orders, _, c_out = w.shape
    b_vec = jnp.asarray(bias, f32).reshape(1, c_out)
    g = jnp.asarray(graph, f32)

    bm = math.gcd(n, 256)
    g_bf, dsc = pl.pallas_call(
        _prep_body,
        out_shape=(jax.ShapeDtypeStruct((n, n), bf16),
                   jax.ShapeDtypeStruct((n, 1), f32)),
        grid=(n // bm,),
        in_specs=[pl.BlockSpec((bm, n), lambda i: (i, 0))],
        out_specs=(pl.BlockSpec((bm, n), lambda i: (i, 0)),
                   pl.BlockSpec((bm, 1), lambda i: (i, 0))),
        compiler_params=pltpu.CompilerParams(
            dimension_semantics=("arbitrary",)),
    )(g)

    nb = batch // 2
    w_bd = jnp.einsum('be,kcd->kbced', jnp.eye(nb, dtype=f32), w)
    w_bd = w_bd.reshape(n_orders, nb * c_in, nb * c_out).astype(bf16)
    b_bd = jnp.tile(b_vec, (1, nb))

    x2d = x.transpose(1, 0, 2).reshape(n, batch * c_in)

    out = pl.pallas_call(
        functools.partial(_cheb_body, n_orders=n_orders, nb=nb, c_out=c_out),
        out_shape=jax.ShapeDtypeStruct((batch, n, c_out), f32),
        grid=(2,),
        in_specs=[
            pl.BlockSpec((n, n), lambda c: (0, 0)),
            pl.BlockSpec((n, 1), lambda c: (0, 0)),
            pl.BlockSpec((n, nb * c_in), lambda c: (0, c)),
            pl.BlockSpec((n_orders, nb * c_in, nb * c_out),
                         lambda c: (0, 0, 0)),
            pl.BlockSpec((1, nb * c_out), lambda c: (0, 0)),
        ],
        out_specs=pl.BlockSpec((nb, n, c_out), lambda c: (c, 0, 0)),
        compiler_params=pltpu.CompilerParams(
            dimension_semantics=("arbitrary",),
            vmem_limit_bytes=56 * 1024 * 1024,
        ),
    )(g_bf, dsc, x2d, w_bd, b_bd)

    return out


# speedup vs baseline: 3.1200x; 1.3984x over previous
import functools
import math

import jax
import jax.numpy as jnp
from jax.experimental import pallas as pl
from jax.experimental.pallas import tpu as pltpu


def _cheb_body(g_ref, x_ref, w_ref, b_ref, out_ref, g8_ref, dsc_ref, *,
               np_steps, bm, n_orders, nb, c_out):
    f32 = jnp.float32
    bf16 = jnp.bfloat16
    i = pl.program_id(0)

    @pl.when(i < np_steps)
    def _prep():
        gf = g_ref[...]
        row0 = pl.multiple_of(i * bm, bm)
        g8_ref[pl.ds(row0, bm), :] = gf.astype(jnp.float8_e4m3fn)
        dsc_ref[pl.ds(row0, bm), :] = jax.lax.rsqrt(
            jnp.sum(gf, axis=1, keepdims=True))

    @pl.when(i >= np_steps)
    def _main():
        g = g8_ref[...]
        dsc = dsc_ref[...]
        dsc_up = dsc * 32.0
        dsc_dn = dsc * (1.0 / 32.0)

        def s_matvec(xv):
            xs = (dsc_up * xv).astype(jnp.float8_e4m3fn)
            return dsc_dn * jnp.dot(g, xs, preferred_element_type=f32)

        x0 = jnp.concatenate([x_ref[b] for b in range(nb)], axis=1)
        x1 = x0 - s_matvec(x0)
        xcat = [x0.astype(bf16), x1.astype(bf16)]
        xm2, xm1 = x0, x1
        for k in range(2, n_orders):
            xk = 2.0 * (xm1 - s_matvec(xm1)) - xm2
            xcat.append(xk.astype(bf16))
            xm2, xm1 = xm1, xk

        out = jnp.dot(jnp.concatenate(xcat, axis=1), w_ref[...],
                      preferred_element_type=f32) + b_ref[...]

        for b in range(nb):
            out_ref[b] = out[:, b * c_out:(b + 1) * c_out]


def kernel(inputs, graph, weight, bias):
    f32 = jnp.float32
    bf16 = jnp.bfloat16

    x = jnp.asarray(inputs, f32)
    batch, n, c_in = x.shape
    w = jnp.asarray(weight, f32)[:, 0]
    n_orders, _, c_out = w.shape
    b_vec = jnp.asarray(bias, f32).reshape(1, c_out)
    g = jnp.asarray(graph, f32)

    nb = batch // 2
    w_bd = jnp.einsum('be,kcd->kbced', jnp.eye(nb, dtype=f32), w)
    w_bd = w_bd.reshape(n_orders * nb * c_in, nb * c_out).astype(bf16)
    b_bd = jnp.tile(b_vec, (1, nb))

    bm = math.gcd(n, 1024)
    np_steps = n // bm

    out = pl.pallas_call(
        functools.partial(_cheb_body, np_steps=np_steps, bm=bm,
                          n_orders=n_orders, nb=nb, c_out=c_out),
        out_shape=jax.ShapeDtypeStruct((batch, n, c_out), f32),
        grid=(np_steps + 2,),
        in_specs=[
            pl.BlockSpec((bm, n),
                         lambda i: (jnp.minimum(i, np_steps - 1), 0)),
            pl.BlockSpec((nb, n, c_in),
                         lambda i: (jnp.maximum(i - np_steps, 0), 0, 0)),
            pl.BlockSpec((n_orders * nb * c_in, nb * c_out),
                         lambda i: (0, 0)),
            pl.BlockSpec((1, nb * c_out), lambda i: (0, 0)),
        ],
        out_specs=pl.BlockSpec((nb, n, c_out),
                               lambda i: (jnp.maximum(i - np_steps, 0),
                                          0, 0)),
        scratch_shapes=[
            pltpu.VMEM((n, n), jnp.float8_e4m3fn),
            pltpu.VMEM((n, 1), f32),
        ],
        compiler_params=pltpu.CompilerParams(
            dimension_semantics=("arbitrary",),
            vmem_limit_bytes=56 * 1024 * 1024,
        ),
    )(g, x, w_bd, b_bd)

    return out
